# Initial kernel scaffold; baseline (speedup 1.0000x reference)
import jax
import jax.numpy as jnp
from jax.experimental import pallas as pl


def kernel(x, edge_index, Wp1, bp1, Wp2, bp2, Wt1, bt1, Wt2, bt2):
    raise NotImplementedError("write your pallas kernel here")


# speedup vs baseline: 8.7112x; 8.7112x over previous
import functools

import jax
import jax.numpy as jnp
from jax import lax
from jax.experimental import pallas as pl
from jax.experimental.pallas import tpu as pltpu
from jax.experimental.pallas import tpu_sc as plsc

N = 10000
D = 128
H = 128
OUT = 64
E = 320000

NC = 2
NS = 16
NW = NC * NS
EPW = E // NW
CHUNK = 80
NCHUNK = EPW // CHUNK
NP = 10240
RPT = NP // NS
ZROWS = 128

_mesh = plsc.VectorSubcoreMesh(
    core_axis_name="c", subcore_axis_name="s", num_cores=NC, num_subcores=NS)


def _zero_2d(buf, nrows, width):
    def body(i, carry):
        for j in range(width // 16):
            buf[i, pl.ds(j * 16, 16)] = jnp.zeros((16,), jnp.float32)
        return carry
    lax.fori_loop(0, nrows, body, 0)


def _zero_1d(buf, n):
    def body(i, carry):
        buf[pl.ds(i * 16, 16)] = jnp.zeros((16,), jnp.float32)
        return carry
    lax.fori_loop(0, n // 16, body, 0)


@functools.partial(
    pl.kernel,
    out_type=(jax.ShapeDtypeStruct((NC * NP,), jnp.float32),
              jax.ShapeDtypeStruct((NC * NP,), jnp.float32)),
    mesh=_mesh,
    scratch_types=[
        pltpu.VMEM((CHUNK,), jnp.int32),
        pltpu.VMEM((CHUNK,), jnp.int32),
        pltpu.VMEM((CHUNK,), jnp.float32),
        pltpu.VMEM((RPT,), jnp.float32),
        pltpu.VMEM_SHARED((NP,), jnp.float32),
        pltpu.VMEM_SHARED((NP,), jnp.float32),
    ],
)
def _sc_degree(src_hbm, dst_hbm, degO_hbm, degI_hbm,
               sidx, didx, onesb, zb, accO, accI):
    c = lax.axis_index("c")
    s = lax.axis_index("s")
    wid = s * NC + c

    _zero_1d(zb, RPT)
    _zero_1d(onesb, CHUNK)

    def ones_body(i, carry):
        onesb[pl.ds(i * 16, 16)] = jnp.full((16,), 1.0, jnp.float32)
        return carry
    lax.fori_loop(0, CHUNK // 16, ones_body, 0)

    pltpu.sync_copy(zb, accO.at[pl.ds(s * RPT, RPT)])
    pltpu.sync_copy(zb, accI.at[pl.ds(s * RPT, RPT)])
    plsc.subcore_barrier()

    def chunk_body(j, carry):
        eoff = pl.multiple_of(wid * EPW + j * CHUNK, 8)
        pltpu.sync_copy(src_hbm.at[pl.ds(eoff, CHUNK)], sidx)
        pltpu.sync_copy(dst_hbm.at[pl.ds(eoff, CHUNK)], didx)
        pltpu.sync_copy(onesb, accO.at[sidx], add=True)
        pltpu.sync_copy(onesb, accI.at[didx], add=True)
        return carry
    lax.fori_loop(0, NCHUNK, chunk_body, 0)

    plsc.subcore_barrier()
    pltpu.sync_copy(accO.at[pl.ds(s * RPT, RPT)],
                    degO_hbm.at[pl.ds(c * NP + s * RPT, RPT)])
    pltpu.sync_copy(accI.at[pl.ds(s * RPT, RPT)],
                    degI_hbm.at[pl.ds(c * NP + s * RPT, RPT)])


@functools.partial(
    pl.kernel,
    out_type=(jax.ShapeDtypeStruct((NC * NP, D), jnp.float32),
              jax.ShapeDtypeStruct((NC * NP,), jnp.float32)),
    mesh=_mesh,
    scratch_types=[
        pltpu.VMEM((CHUNK,), jnp.int32),
        pltpu.VMEM((CHUNK,), jnp.int32),
        pltpu.VMEM((CHUNK, D), jnp.float32),
        pltpu.VMEM((CHUNK,), jnp.float32),
        pltpu.VMEM((ZROWS, D), jnp.float32),
        pltpu.VMEM_SHARED((NP, D), jnp.float32),
        pltpu.VMEM_SHARED((NP,), jnp.float32),
        pltpu.SemaphoreType.DMA,
        pltpu.SemaphoreType.DMA,
    ],
)
def _sc_pass1(xt_hbm, w_hbm, src_hbm, dst_hbm, px_hbm, ps_hbm,
              sidx, didx, xrows, wbuf, zbuf, accx, accs, sem1, sem2):
    c = lax.axis_index("c")
    s = lax.axis_index("s")
    wid = s * NC + c

    _zero_2d(zbuf, ZROWS, D)
    _zero_1d(wbuf, CHUNK)
    for k in range(RPT // ZROWS):
        pltpu.sync_copy(zbuf, accx.at[pl.ds(s * RPT + k * ZROWS, ZROWS)])
    for k in range(RPT // CHUNK):
        pltpu.sync_copy(wbuf, accs.at[pl.ds(s * RPT + k * CHUNK, CHUNK)])
    plsc.subcore_barrier()

    def chunk_body(j, carry):
        eoff = pl.multiple_of(wid * EPW + j * CHUNK, 8)
        pltpu.sync_copy(src_hbm.at[pl.ds(eoff, CHUNK)], sidx)
        pltpu.sync_copy(dst_hbm.at[pl.ds(eoff, CHUNK)], didx)
        g1 = pltpu.async_copy(xt_hbm.at[sidx], xrows, sem1)
        g2 = pltpu.async_copy(w_hbm.at[sidx], wbuf, sem2)
        g1.wait()
        g2.wait()
        pltpu.sync_copy(xrows, accx.at[didx], add=True)
        pltpu.sync_copy(wbuf, accs.at[didx], add=True)
        return carry
    lax.fori_loop(0, NCHUNK, chunk_body, 0)

    plsc.subcore_barrier()
    for k in range(RPT // ZROWS):
        pltpu.sync_copy(accx.at[pl.ds(s * RPT + k * ZROWS, ZROWS)],
                        px_hbm.at[pl.ds(c * NP + s * RPT + k * ZROWS, ZROWS)])
    pltpu.sync_copy(accs.at[pl.ds(s * RPT, RPT)],
                    ps_hbm.at[pl.ds(c * NP + s * RPT, RPT)])


@functools.partial(
    pl.kernel,
    out_type=jax.ShapeDtypeStruct((NC * NP, D), jnp.float32),
    mesh=_mesh,
    scratch_types=[
        pltpu.VMEM((CHUNK,), jnp.int32),
        pltpu.VMEM((CHUNK,), jnp.int32),
        pltpu.VMEM((CHUNK, D), jnp.float32),
        pltpu.VMEM((ZROWS, D), jnp.float32),
        pltpu.VMEM_SHARED((NP, D), jnp.float32),
        pltpu.SemaphoreType.DMA,
    ],
)
def _sc_pass2(zt_hbm, src_hbm, dst_hbm, pz_hbm,
              sidx, didx, zrows, zbuf, accz, sem1):
    c = lax.axis_index("c")
    s = lax.axis_index("s")
    wid = s * NC + c

    _zero_2d(zbuf, ZROWS, D)
    for k in range(RPT // ZROWS):
        pltpu.sync_copy(zbuf, accz.at[pl.ds(s * RPT + k * ZROWS, ZROWS)])
    plsc.subcore_barrier()

    def chunk_body(j, carry):
        eoff = pl.multiple_of(wid * EPW + j * CHUNK, 8)
        pltpu.sync_copy(src_hbm.at[pl.ds(eoff, CHUNK)], sidx)
        pltpu.sync_copy(dst_hbm.at[pl.ds(eoff, CHUNK)], didx)
        pltpu.async_copy(zt_hbm.at[sidx], zrows, sem1).wait()
        pltpu.sync_copy(zrows, accz.at[didx], add=True)
        return carry
    lax.fori_loop(0, NCHUNK, chunk_body, 0)

    plsc.subcore_barrier()
    for k in range(RPT // ZROWS):
        pltpu.sync_copy(accz.at[pl.ds(s * RPT + k * ZROWS, ZROWS)],
                        pz_hbm.at[pl.ds(c * NP + s * RPT + k * ZROWS, ZROWS)])


_BLK = 1000
_NBLK = N // _BLK


def _tc_prep_body(degO0_ref, degO1_ref, x_ref, xt_ref, w_ref):
    deg = degO0_ref[...] + degO1_ref[...]
    inv = jnp.where(deg > 0.0, 1.0 / jnp.sqrt(deg), 0.0)
    xt_ref[...] = x_ref[...] * inv
    w_ref[...] = inv


def _tc_prep(degO0, degO1, x):
    return pl.pallas_call(
        _tc_prep_body,
        grid=(_NBLK,),
        in_specs=[
            pl.BlockSpec((_BLK, 1), lambda i: (i, 0)),
            pl.BlockSpec((_BLK, 1), lambda i: (i, 0)),
            pl.BlockSpec((_BLK, D), lambda i: (i, 0)),
        ],
        out_specs=[
            pl.BlockSpec((_BLK, D), lambda i: (i, 0)),
            pl.BlockSpec((_BLK, 1), lambda i: (i, 0)),
        ],
        out_shape=[
            jax.ShapeDtypeStruct((N, D), jnp.float32),
            jax.ShapeDtypeStruct((N, 1), jnp.float32),
        ],
    )(degO0, degO1, x)


def _tc_dense_body(px0_ref, px1_ref, ps0_ref, ps1_ref,
                   degI0_ref, degI1_ref, w_ref,
                   Wt1_ref, Wp1_ref, bt1_ref, bp1_ref,
                   Wt2_ref, Wp2_ref, bd2_ref, zt_ref):
    aggx = px0_ref[...] + px1_ref[...]
    din = degI0_ref[...] + degI1_ref[...]
    dii = jnp.where(din > 0.0, 1.0 / jnp.sqrt(din), 0.0)
    xa = aggx * dii
    ones_a = (ps0_ref[...] + ps1_ref[...]) * dii
    ht = jnp.maximum(
        jnp.dot(xa, Wt1_ref[...], preferred_element_type=jnp.float32)
        + ones_a * bt1_ref[...], 0.0)
    hp = jnp.maximum(
        jnp.dot(xa, Wp1_ref[...], preferred_element_type=jnp.float32)
        + ones_a * bp1_ref[...], 0.0)
    z = (jnp.dot(ht, Wt2_ref[...], preferred_element_type=jnp.float32)
         - jnp.dot(hp, Wp2_ref[...], preferred_element_type=jnp.float32)
         + bd2_ref[...])
    zz = z * w_ref[...]
    zt_ref[...] = jnp.concatenate(
        [zz, jnp.zeros((_BLK, D - OUT), jnp.float32)], axis=1)


def _tc_dense(px0, px1, ps0, ps1, degI0, degI1, w,
              Wt1, Wp1, bt1, bp1, Wt2, Wp2, bd2):
    full = lambda a, b: pl.BlockSpec((a, b), lambda i: (0, 0))
    col = pl.BlockSpec((_BLK, 1), lambda i: (i, 0))
    return pl.pallas_call(
        _tc_dense_body,
        grid=(_NBLK,),
        in_specs=[
            pl.BlockSpec((_BLK, D), lambda i: (i, 0)),
            pl.BlockSpec((_BLK, D), lambda i: (i, 0)),
            col, col, col, col, col,
            full(D, H), full(D, H), full(1, H), full(1, H),
            full(H, OUT), full(H, OUT), full(1, OUT),
        ],
        out_specs=pl.BlockSpec((_BLK, D), lambda i: (i, 0)),
        out_shape=jax.ShapeDtypeStruct((N, D), jnp.float32),
    )(px0, px1, ps0, ps1, degI0, degI1, w,
      Wt1, Wp1, bt1, bp1, Wt2, Wp2, bd2)


def _tc_loss_body(pz0_ref, pz1_ref, degI0_ref, degI1_ref, out_ref):
    i = pl.program_id(0)
    d = pz0_ref[...] + pz1_ref[...]
    din = degI0_ref[...] + degI1_ref[...]
    dii = jnp.where(din > 0.0, 1.0 / jnp.sqrt(din), 0.0)
    dd = d * dii
    part = jnp.sum(dd * dd).reshape(1, 1)

    @pl.when(i == 0)
    def _():
        out_ref[...] = jnp.zeros((1, 1), jnp.float32)

    out_ref[...] += part

    @pl.when(i == _NBLK - 1)
    def _():
        out_ref[...] = out_ref[...] * (1.0 / (N * OUT))


def _tc_loss(pz0, pz1, degI0, degI1):
    col = pl.BlockSpec((_BLK, 1), lambda i: (i, 0))
    return pl.pallas_call(
        _tc_loss_body,
        grid=(_NBLK,),
        in_specs=[
            pl.BlockSpec((_BLK, D), lambda i: (i, 0)),
            pl.BlockSpec((_BLK, D), lambda i: (i, 0)),
            col, col,
        ],
        out_specs=pl.BlockSpec((1, 1), lambda i: (0, 0)),
        out_shape=jax.ShapeDtypeStruct((1, 1), jnp.float32),
    )(pz0, pz1, degI0, degI1)


def kernel(x, edge_index, Wp1, bp1, Wp2, bp2, Wt1, bt1, Wt2, bt2):
    src = edge_index[0]
    dst = edge_index[1]

    degO, degI = _sc_degree(src, dst)
    degO0 = degO[:N].reshape(N, 1)
    degO1 = degO[NP:NP + N].reshape(N, 1)
    degI0 = degI[:N].reshape(N, 1)
    degI1 = degI[NP:NP + N].reshape(N, 1)

    xt, w2d = _tc_prep(degO0, degO1, x)
    w = w2d.reshape(N)

    px, ps = _sc_pass1(xt, w, src, dst)
    ps0 = ps[:N].reshape(N, 1)
    ps1 = ps[NP:NP + N].reshape(N, 1)

    bt1r = bt1.reshape(1, H)
    bp1r = bp1.reshape(1, H)
    bd2 = (bt2 - bp2).reshape(1, OUT)
    zt = _tc_dense(px[:N], px[NP:NP + N], ps0, ps1, degI0, degI1, w2d,
                   Wt1, Wp1, bt1r, bp1r, Wt2, Wp2, bd2)

    pz = _sc_pass2(zt, src, dst)
    loss = _tc_loss(pz[:N], pz[NP:NP + N], degI0, degI1)
    return loss[0, 0]
